# baseline (speedup 1.0000x reference)
import jax
import jax.numpy as jnp
from jax import lax
from jax.experimental import pallas as pl
from jax.experimental.pallas import tpu as pltpu
from jax.experimental.pallas import tpu_sc as plsc

B = 16384
C = 2
N_IN = 234
N_OUT = 256
ROW_IN = C * N_IN
ROW_OUT = C * N_OUT
NUM_WORKERS = 32
BPW = B // NUM_WORKERS
CB = 64
NCHUNK = BPW // CB
ROWS = CB * C
PAD = 8


def _sc_body(in_hbm, out_hbm, in_v, out_v):
    wid = lax.axis_index("s") * 2 + lax.axis_index("c")
    e0 = wid * BPW

    iota = lax.broadcasted_iota(jnp.int32, (16,), 0)
    zero = jnp.zeros((16,), jnp.float32)

    def assemble_row(r, _):
        base = PAD + r * N_IN
        obase = r * N_OUT
        pilot = (1 - (r % 2)).astype(jnp.float32)

        def ld(off):
            return in_v[pl.ds(base + off, 16)]

        def st(g, v):
            out_v[pl.ds(obase + g * 16, 16)] = v

        a = ld(-7)
        st(0, jnp.where(iota == 6, pilot, jnp.where(iota <= 5, 0.0, a)))
        st(1, ld(9))
        st(2, ld(25))
        a = ld(41)
        b = ld(40)
        st(3, jnp.where(iota == 11, pilot, jnp.where(iota <= 10, a, b)))
        st(4, ld(56))
        st(5, ld(72))
        st(6, ld(88))
        a = ld(104)
        st(7, jnp.where(iota <= 12, a, 0.0))
        a = ld(113)
        st(8, jnp.where(iota >= 4, a, 0.0))
        st(9, ld(129))
        st(10, ld(145))
        st(11, ld(161))
        a = ld(177)
        b = ld(176)
        st(12, jnp.where(iota == 5, pilot, jnp.where(iota <= 4, a, b)))
        st(13, ld(192))
        st(14, ld(208))
        a = ld(224)
        st(15, jnp.where(iota == 10, pilot, jnp.where(iota <= 9, a, 0.0)))
        return 0

    def chunk(i, _):
        start = (e0 + i * CB) * ROW_IN
        pltpu.sync_copy(
            in_hbm.at[pl.ds(start, CB * ROW_IN)],
            in_v.at[pl.ds(PAD, CB * ROW_IN)],
        )
        lax.fori_loop(0, ROWS, assemble_row, 0)
        pltpu.sync_copy(
            out_v,
            out_hbm.at[pl.ds((e0 + i * CB) * ROW_OUT, CB * ROW_OUT)],
        )
        return 0

    lax.fori_loop(0, NCHUNK, chunk, 0)


@jax.jit
def kernel(data_freq):
    mesh = plsc.VectorSubcoreMesh(core_axis_name="c", subcore_axis_name="s")
    out_flat = pl.kernel(
        _sc_body,
        out_type=jax.ShapeDtypeStruct((B * ROW_OUT,), jnp.float32),
        mesh=mesh,
        scratch_types=[
            pltpu.VMEM((PAD + CB * ROW_IN + PAD,), jnp.float32),
            pltpu.VMEM((CB * ROW_OUT,), jnp.float32),
        ],
    )(data_freq.reshape(-1))
    return out_flat.reshape(B, C, N_OUT)


# speedup vs baseline: 1.0988x; 1.0988x over previous
import jax
import jax.numpy as jnp
from jax import lax
from jax.experimental import pallas as pl
from jax.experimental.pallas import tpu as pltpu
from jax.experimental.pallas import tpu_sc as plsc

B = 16384
C = 2
N_IN = 234
N_OUT = 256
ROW_IN = C * N_IN
ROW_OUT = C * N_OUT
NUM_WORKERS = 32
BPW = B // NUM_WORKERS
CB = 64
NCHUNK = BPW // CB
PAD = 8

IN_WORDS = CB * ROW_IN
OUT_WORDS = CB * ROW_OUT


def _sc_body(in_hbm, out_hbm, in_a, in_b, out_a, out_b,
             sin_a, sin_b, sout_a, sout_b):
    wid = lax.axis_index("s") * 2 + lax.axis_index("c")
    e0 = wid * BPW

    iota = lax.broadcasted_iota(jnp.int32, (16,), 0)
    zeros = jnp.zeros((16,), jnp.float32)
    ones = jnp.ones((16,), jnp.float32)
    m_le5 = iota <= 5
    m_le6 = iota <= 6
    m_eq6 = iota == 6
    m_le10 = iota <= 10
    m_eq11 = iota == 11
    m_le12 = iota <= 12
    m_ge4 = iota >= 4
    m_le4 = iota <= 4
    m_eq5 = iota == 5
    m_le9 = iota <= 9
    m_eq10 = iota == 10
    cv_g0 = jnp.where(m_eq6, 1.0, 0.0).astype(jnp.float32)

    def in_slice(i):
        return in_hbm.at[pl.ds((e0 + i * CB) * ROW_IN, IN_WORDS)]

    def out_slice(i):
        return out_hbm.at[pl.ds((e0 + i * CB) * ROW_OUT, OUT_WORDS)]

    def assemble(in_v, out_v):
        def row(rb, _):
            for c in (0, 1):
                base = PAD + rb * ROW_IN + c * N_IN
                obase = rb * ROW_OUT + c * N_OUT
                pvec = ones if c == 0 else zeros
                g0c = cv_g0 if c == 0 else zeros

                def ld(off):
                    return in_v[pl.ds(base + off, 16)]

                def st(g, v):
                    out_v[pl.ds(obase + g * 16, 16)] = v

                st(0, jnp.where(m_le6, g0c, ld(-7)))
                st(1, ld(9))
                st(2, ld(25))
                st(3, jnp.where(m_eq11, pvec, jnp.where(m_le10, ld(41), ld(40))))
                st(4, ld(56))
                st(5, ld(72))
                st(6, ld(88))
                st(7, jnp.where(m_le12, ld(104), 0.0))
                st(8, jnp.where(m_ge4, ld(113), 0.0))
                st(9, ld(129))
                st(10, ld(145))
                st(11, ld(161))
                st(12, jnp.where(m_eq5, pvec, jnp.where(m_le4, ld(177), ld(176))))
                st(13, ld(192))
                st(14, ld(208))
                st(15, jnp.where(m_eq10, pvec, jnp.where(m_le9, ld(224), 0.0)))
            return 0

        lax.fori_loop(0, CB, row, 0)

    bufs = ((in_a, sin_a, out_a, sout_a), (in_b, sin_b, out_b, sout_b))

    def in_start(i, ph):
        in_v, sin = bufs[ph][0], bufs[ph][1]
        pltpu.async_copy(in_slice(i), in_v.at[pl.ds(PAD, IN_WORDS)], sin)

    def in_wait(i, ph):
        in_v, sin = bufs[ph][0], bufs[ph][1]
        pltpu.make_async_copy(in_slice(i), in_v.at[pl.ds(PAD, IN_WORDS)], sin).wait()

    def out_start(i, ph):
        out_v, sout = bufs[ph][2], bufs[ph][3]
        pltpu.async_copy(out_v, out_slice(i), sout)

    def out_wait(i, ph):
        out_v, sout = bufs[ph][2], bufs[ph][3]
        pltpu.make_async_copy(out_v, out_slice(i), sout).wait()

    in_start(0, 0)

    def step(k, _):
        c0 = 2 * k
        in_start(c0 + 1, 1)
        in_wait(c0, 0)

        @pl.when(k > 0)
        def _():
            out_wait(c0 - 2, 0)

        assemble(in_a, out_a)
        out_start(c0, 0)

        @pl.when(k < NCHUNK // 2 - 1)
        def _():
            in_start(c0 + 2, 0)

        in_wait(c0 + 1, 1)

        @pl.when(k > 0)
        def _():
            out_wait(c0 - 1, 1)

        assemble(in_b, out_b)
        out_start(c0 + 1, 1)
        return 0

    lax.fori_loop(0, NCHUNK // 2, step, 0)
    out_wait(NCHUNK - 2, 0)
    out_wait(NCHUNK - 1, 1)


@jax.jit
def kernel(data_freq):
    mesh = plsc.VectorSubcoreMesh(core_axis_name="c", subcore_axis_name="s")
    out_flat = pl.kernel(
        _sc_body,
        out_type=jax.ShapeDtypeStruct((B * ROW_OUT,), jnp.float32),
        mesh=mesh,
        scratch_types=[
            pltpu.VMEM((PAD + IN_WORDS + PAD,), jnp.float32),
            pltpu.VMEM((PAD + IN_WORDS + PAD,), jnp.float32),
            pltpu.VMEM((OUT_WORDS,), jnp.float32),
            pltpu.VMEM((OUT_WORDS,), jnp.float32),
            pltpu.SemaphoreType.DMA,
            pltpu.SemaphoreType.DMA,
            pltpu.SemaphoreType.DMA,
            pltpu.SemaphoreType.DMA,
        ],
    )(data_freq.reshape(-1))
    return out_flat.reshape(B, C, N_OUT)


# speedup vs baseline: 2.5267x; 2.2995x over previous
import jax
import jax.numpy as jnp
from jax import lax
from jax.experimental import pallas as pl
from jax.experimental.pallas import tpu as pltpu
from jax.experimental.pallas import tpu_sc as plsc

B = 16384
C = 2
N_IN = 234
N_OUT = 256
NUM_WORKERS = 32
BPW = B // NUM_WORKERS
CB = 64
NCHUNK = BPW // CB


def _sc_body(in_hbm, out_hbm, in_a, in_b, out_a, out_b,
             sin_a, sin_b, sout_a, sout_b):
    wid = lax.axis_index("s") * 2 + lax.axis_index("c")
    e0 = wid * BPW

    iota = lax.broadcasted_iota(jnp.int32, (16,), 0)
    zeros = jnp.zeros((16,), jnp.float32)
    ones = jnp.ones((16,), jnp.float32)
    m_le6 = iota <= 6
    m_le10 = iota <= 10
    m_eq11 = iota == 11
    m_le12 = iota <= 12
    m_ge4 = iota >= 4
    m_le4 = iota <= 4
    m_eq5 = iota == 5
    m_le9 = iota <= 9
    m_eq10 = iota == 10
    cv_g0 = jnp.where(iota == 6, 1.0, 0.0).astype(jnp.float32)
    idx_g0 = jnp.maximum(iota - 7, 0)
    idx_g15 = jnp.minimum(iota + 6, 15)

    def in_slice(i):
        return in_hbm.at[pl.ds(e0 + i * CB, CB)]

    def out_slice(i):
        return out_hbm.at[pl.ds(e0 + i * CB, CB)]

    def assemble(in_v, out_v):
        @plsc.parallel_loop(0, CB, unroll=2)
        def _(rb):
            for c in (0, 1):
                pvec = ones if c == 0 else zeros
                g0c = cv_g0 if c == 0 else zeros

                def ld(off):
                    return in_v[rb, c, pl.ds(off, 16)]

                def st(g, v):
                    out_v[rb, c, pl.ds(g * 16, 16)] = v

                a = jnp.take_along_axis(ld(0), idx_g0, axis=0)
                st(0, jnp.where(m_le6, g0c, a))
                st(1, ld(9))
                st(2, ld(25))
                st(3, jnp.where(m_eq11, pvec, jnp.where(m_le10, ld(41), ld(40))))
                st(4, ld(56))
                st(5, ld(72))
                st(6, ld(88))
                st(7, jnp.where(m_le12, ld(104), 0.0))
                st(8, jnp.where(m_ge4, ld(113), 0.0))
                st(9, ld(129))
                st(10, ld(145))
                st(11, ld(161))
                st(12, jnp.where(m_eq5, pvec, jnp.where(m_le4, ld(177), ld(176))))
                st(13, ld(192))
                st(14, ld(208))
                a = jnp.take_along_axis(ld(218), idx_g15, axis=0)
                st(15, jnp.where(m_eq10, pvec, jnp.where(m_le9, a, 0.0)))

    bufs = ((in_a, sin_a, out_a, sout_a), (in_b, sin_b, out_b, sout_b))

    def in_start(i, ph):
        pltpu.async_copy(in_slice(i), bufs[ph][0], bufs[ph][1])

    def in_wait(i, ph):
        pltpu.make_async_copy(in_slice(i), bufs[ph][0], bufs[ph][1]).wait()

    def out_start(i, ph):
        pltpu.async_copy(bufs[ph][2], out_slice(i), bufs[ph][3])

    def out_wait(i, ph):
        pltpu.make_async_copy(bufs[ph][2], out_slice(i), bufs[ph][3]).wait()

    in_start(0, 0)

    def step(k, _):
        c0 = 2 * k
        in_start(c0 + 1, 1)
        in_wait(c0, 0)

        @pl.when(k > 0)
        def _():
            out_wait(c0 - 2, 0)

        assemble(in_a, out_a)
        out_start(c0, 0)

        @pl.when(k < NCHUNK // 2 - 1)
        def _():
            in_start(c0 + 2, 0)

        in_wait(c0 + 1, 1)

        @pl.when(k > 0)
        def _():
            out_wait(c0 - 1, 1)

        assemble(in_b, out_b)
        out_start(c0 + 1, 1)
        return 0

    lax.fori_loop(0, NCHUNK // 2, step, 0)
    out_wait(NCHUNK - 2, 0)
    out_wait(NCHUNK - 1, 1)


@jax.jit
def kernel(data_freq):
    mesh = plsc.VectorSubcoreMesh(core_axis_name="c", subcore_axis_name="s")
    return pl.kernel(
        _sc_body,
        out_type=jax.ShapeDtypeStruct((B, C, N_OUT), jnp.float32),
        mesh=mesh,
        scratch_types=[
            pltpu.VMEM((CB, C, N_IN), jnp.float32),
            pltpu.VMEM((CB, C, N_IN), jnp.float32),
            pltpu.VMEM((CB, C, N_OUT), jnp.float32),
            pltpu.VMEM((CB, C, N_OUT), jnp.float32),
            pltpu.SemaphoreType.DMA,
            pltpu.SemaphoreType.DMA,
            pltpu.SemaphoreType.DMA,
            pltpu.SemaphoreType.DMA,
        ],
    )(data_freq)


# speedup vs baseline: 3.4520x; 1.3662x over previous
import jax
import jax.numpy as jnp
from jax import lax
from jax.experimental import pallas as pl
from jax.experimental.pallas import tpu as pltpu
from jax.experimental.pallas import tpu_sc as plsc

B = 16384
C = 2
N_IN = 234
N_OUT = 256
NUM_WORKERS = 32
KPW = N_OUT // NUM_WORKERS
CW = 2048
ROW_BYTES_ELEMS = C * B


def _sc_body(in_hbm, out_hbm, cbuf, rowbuf, sem):
    wid = lax.axis_index("s") * 2 + lax.axis_index("c")

    ones = jnp.ones((16,), jnp.float32)
    zeros = jnp.zeros((16,), jnp.float32)

    def fill(i, _):
        cbuf[0, pl.ds(i * 16, 16)] = ones
        cbuf[1, pl.ds(i * 16, 16)] = zeros
        cbuf[2, pl.ds(i * 16, 16)] = zeros
        cbuf[3, pl.ds(i * 16, 16)] = zeros
        return 0

    lax.fori_loop(0, CW // 16, fill, 0)

    for j in range(KPW):
        k = wid + NUM_WORKERS * j
        in_seg = (
            ((k >= 7) & (k <= 58))
            | ((k >= 60) & (k <= 124))
            | ((k >= 132) & (k <= 196))
            | ((k >= 198) & (k <= 249))
        )
        shift = (
            7
            + (k >= 60).astype(jnp.int32)
            + 7 * (k >= 132).astype(jnp.int32)
            + (k >= 198).astype(jnp.int32)
        )
        k_in = jnp.maximum(k - shift, 0)
        is_pilot = (k == 6) | (k == 59) | (k == 197) | (k == 250)
        r0 = jnp.where(is_pilot, 0, 2)

        @pl.when(in_seg)
        def _():
            pltpu.sync_copy(in_hbm.at[k_in], rowbuf)
            pltpu.sync_copy(rowbuf, out_hbm.at[k])

        @pl.when(jnp.logical_not(in_seg))
        def _():
            for o in range(0, B, CW):
                pltpu.sync_copy(
                    cbuf.at[pl.ds(r0, 2), :],
                    out_hbm.at[k, :, pl.ds(o, CW)],
                )


def _sc_call(xT):
    mesh = plsc.VectorSubcoreMesh(core_axis_name="c", subcore_axis_name="s")
    return pl.kernel(
        _sc_body,
        out_type=jax.ShapeDtypeStruct((N_OUT, C, B), jnp.float32),
        mesh=mesh,
        scratch_types=[
            pltpu.VMEM((4, CW), jnp.float32),
            pltpu.VMEM((C, B), jnp.float32),
            pltpu.SemaphoreType.DMA,
        ],
    )(xT)


@jax.jit
def kernel(data_freq):
    xT = jnp.transpose(data_freq, (2, 1, 0))
    outT = _sc_call(xT)
    return jnp.transpose(outT, (2, 1, 0))
